# baseline (speedup 1.0000x reference)
import functools

import jax
import jax.numpy as jnp
from jax import lax
from jax.experimental import pallas as pl
from jax.experimental.pallas import tpu as pltpu
from jax.experimental.pallas import tpu_sc as plsc

BATCH = 1024
HIST = 200
BUCKETS = 1000000
D = 48
DW = D // 2
DIM = 512
TOK = BATCH * HIST

NC = 2
NS = 16
NW = NC * NS
TPW = TOK // NW
GCHUNK = 128
OCHUNK = 1280
NGATHER = OCHUNK // GCHUNK
NOUT = TPW // OCHUNK

BT = 2048
G = TOK // BT


def _sc_hash_gather(idx_flat, idx_prev, tbl24):
    mesh = plsc.VectorSubcoreMesh(core_axis_name="c", subcore_axis_name="s")

    @functools.partial(
        pl.kernel,
        mesh=mesh,
        out_type=jax.ShapeDtypeStruct((TOK, 128), jnp.int32),
        scratch_types=[
            pltpu.VMEM((TPW,), jnp.int32),
            pltpu.VMEM((TPW,), jnp.int32),
            pltpu.VMEM((TPW,), jnp.int32),
            pltpu.VMEM((OCHUNK, DW), jnp.int32),
            pltpu.VMEM((OCHUNK, DW), jnp.int32),
            pltpu.SemaphoreType.DMA,
            pltpu.SemaphoreType.DMA,
        ],
        compiler_params=pltpu.CompilerParams(use_tc_tiling_on_sc=False),
    )
    def k(idx_hbm, prev_hbm, tbl_hbm, rows_hbm,
          idx_v, prev_v, big_v, buf_a, buf_b, sem_a, sem_b):
        wid = lax.axis_index("s") * NC + lax.axis_index("c")
        base = wid * TPW
        pltpu.sync_copy(idx_hbm.at[pl.ds(base, TPW)], idx_v)
        pltpu.sync_copy(prev_hbm.at[pl.ds(base, TPW)], prev_v)

        lanes = lax.iota(jnp.int32, 16)

        def hash_body(i, _):
            off = i * 16
            pos = off + lanes
            cur = idx_v[pl.ds(off, 16)]
            prev = prev_v[pl.ds(off, 16)]
            b = (prev * 10007 + cur) % BUCKETS
            b = jnp.where(pos % HIST == 0, 0, b)
            big_v[pl.ds(off, 16)] = b
            return 0

        lax.fori_loop(0, TPW // 16, hash_body, 0)

        bufs = (buf_a, buf_b)
        sems = (sem_a, sem_b)

        def fire(c, buf, sem):
            cps = []
            for j in range(NGATHER):
                srow = c * OCHUNK + j * GCHUNK
                cps.append(pltpu.async_copy(
                    tbl_hbm.at[big_v.at[pl.ds(srow, GCHUNK)]],
                    buf.at[pl.ds(j * GCHUNK, GCHUNK)],
                    sem))
            return cps

        inflight = {0: fire(0, bufs[0], sems[0])}
        for c in range(NOUT):
            if c + 1 < NOUT:
                inflight[c + 1] = fire(c + 1, bufs[(c + 1) % 2], sems[(c + 1) % 2])
            for cp in inflight.pop(c):
                cp.wait()
            pltpu.sync_copy(bufs[c % 2],
                            rows_hbm.at[pl.ds(base + c * OCHUNK, OCHUNK),
                                        pl.ds(0, DW)])

    return k(idx_flat, idx_prev, tbl24)


def _f16_bits_to_f32(v):
    sign = (v & 0x8000) << 16
    mag = v & 0x7FFF
    bits = sign | ((mag << 13) + 0x38000000)
    bits = jnp.where((v & 0x7C00) == 0, sign, bits)
    return lax.bitcast_convert_type(bits, jnp.float32)


def _tc_project(rows32, wt2):
    def body(r_ref, w_ref, o_ref):
        w = r_ref[:, :DW]
        lo = _f16_bits_to_f32(w & 0xFFFF)
        hi = _f16_bits_to_f32((w >> 16) & 0xFFFF)
        acc = jnp.dot(lo, w_ref[:DW], preferred_element_type=jnp.float32)
        acc += jnp.dot(hi, w_ref[DW:], preferred_element_type=jnp.float32)
        o_ref[...] = acc

    return pl.pallas_call(
        body,
        grid=(G,),
        in_specs=[
            pl.BlockSpec((BT, 128), lambda i: (i, 0)),
            pl.BlockSpec((D, DIM), lambda i: (0, 0)),
        ],
        out_specs=pl.BlockSpec((BT, DIM), lambda i: (i, 0)),
        out_shape=jax.ShapeDtypeStruct((TOK, DIM), jnp.float32),
    )(rows32, wt2)


def kernel(idx, table, W):
    idx_flat = idx.reshape(-1)
    idx_prev = jnp.concatenate([idx_flat[:1], idx_flat[:-1]])
    tbl24 = lax.bitcast_convert_type(
        table.reshape(BUCKETS, DW, 2), jnp.int32)
    rows32 = _sc_hash_gather(idx_flat, idx_prev, tbl24)
    wt = W.T
    wt2 = jnp.concatenate([wt[0::2], wt[1::2]], axis=0)
    out = _tc_project(rows32, wt2)
    return out.reshape(BATCH, HIST, DIM)
